# Initial kernel scaffold; baseline (speedup 1.0000x reference)
import jax
import jax.numpy as jnp
from jax.experimental import pallas as pl


def kernel(sessions, pos_emb):
    raise NotImplementedError("write your pallas kernel here")


# speedup vs baseline: 11.8058x; 11.8058x over previous
import jax
import jax.numpy as jnp
from jax.experimental import pallas as pl
from jax.experimental.pallas import tpu as pltpu


def _body(s_ref, p_ref, o_ref, pf_ref):
    @pl.when(pl.program_id(0) == 0)
    def _():
        L = p_ref.shape[0]
        i = jax.lax.broadcasted_iota(jnp.int32, (L, L), 0)
        j = jax.lax.broadcasted_iota(jnp.int32, (L, L), 1)
        rev = (i + j == L - 1).astype(p_ref.dtype)
        pf_ref[...] = jax.lax.dot(rev, p_ref[...],
                                  preferred_element_type=jnp.float32)

    o_ref[...] = s_ref[...] + pf_ref[...][None, :, :]


def kernel(sessions, pos_emb):
    B, L, F = sessions.shape
    BLK_B = 64
    return pl.pallas_call(
        _body,
        grid=(B // BLK_B,),
        in_specs=[
            pl.BlockSpec((BLK_B, L, F), lambda i: (i, 0, 0)),
            pl.BlockSpec((L, F), lambda i: (0, 0)),
        ],
        out_specs=pl.BlockSpec((BLK_B, L, F), lambda i: (i, 0, 0)),
        out_shape=jax.ShapeDtypeStruct((B, L, F), sessions.dtype),
        scratch_shapes=[pltpu.VMEM((L, F), jnp.float32)],
    )(sessions, pos_emb)


# speedup vs baseline: 11.9415x; 1.0115x over previous
import jax
import jax.numpy as jnp
from jax.experimental import pallas as pl
from jax.experimental.pallas import tpu as pltpu


def _body(s_ref, p_ref, o_ref, pf_ref):
    @pl.when(pl.program_id(0) == 0)
    def _():
        L = p_ref.shape[0]
        i = jax.lax.broadcasted_iota(jnp.int32, (L, L), 0)
        j = jax.lax.broadcasted_iota(jnp.int32, (L, L), 1)
        rev = (i + j == L - 1).astype(p_ref.dtype)
        pf_ref[...] = jax.lax.dot(rev, p_ref[...],
                                  preferred_element_type=jnp.float32)

    o_ref[...] = s_ref[...] + pf_ref[...][None, :, :]


def kernel(sessions, pos_emb):
    B, L, F = sessions.shape
    BLK_B = 128
    return pl.pallas_call(
        _body,
        grid=(B // BLK_B,),
        in_specs=[
            pl.BlockSpec((BLK_B, L, F), lambda i: (i, 0, 0)),
            pl.BlockSpec((L, F), lambda i: (0, 0)),
        ],
        out_specs=pl.BlockSpec((BLK_B, L, F), lambda i: (i, 0, 0)),
        out_shape=jax.ShapeDtypeStruct((B, L, F), sessions.dtype),
        scratch_shapes=[pltpu.VMEM((L, F), jnp.float32)],
    )(sessions, pos_emb)
